# baseline (speedup 1.0000x reference)
import functools

import jax
import jax.numpy as jnp
from jax import lax
from jax.experimental import pallas as pl
from jax.experimental.pallas import tpu as pltpu
from jax.experimental.pallas import tpu_sc as plsc

B = 1024
L = 200
D = 16
H = 3 * D

_NC = 2
_NS = 16
_NW = _NC * _NS
_BPW = B // _NW


def _sc_embed_body(content_hbm, pinyin_hbm, stroke_hbm, wt_hbm, pt_hbm, st_hbm,
                   score_hbm, cidx, rows, slab, pidx, sidx, prow, srow, sem):
    wid = lax.axis_index("s") * _NC + lax.axis_index("c")
    base = wid * _BPW
    pltpu.sync_copy(content_hbm.at[pl.ds(base * L, _BPW * L)], cidx)
    pltpu.sync_copy(pinyin_hbm.at[pl.ds(base, _BPW)], pidx)
    pltpu.sync_copy(stroke_hbm.at[pl.ds(base, _BPW)], sidx)
    cp = pltpu.async_copy(pt_hbm.at[pidx], prow, sem)
    cs = pltpu.async_copy(st_hbm.at[sidx], srow, sem)

    def body(i, carry):
        off = pl.multiple_of(i * L, 8)
        c1 = pltpu.async_copy(
            wt_hbm.at[cidx.at[pl.ds(off, 128)]], rows.at[pl.ds(0, 128)], sem)
        c2 = pltpu.async_copy(
            wt_hbm.at[cidx.at[pl.ds(off + 128, L - 128)]],
            rows.at[pl.ds(128, L - 128)], sem)
        c1.wait()
        c2.wait()
        a0, a1, a2, a3 = rows[0], rows[1], rows[2], rows[3]
        for j in range(4, L, 4):
            a0 += rows[j]
            a1 += rows[j + 1]
            a2 += rows[j + 2]
            a3 += rows[j + 3]
        acc = (a0 + a1) + (a2 + a3)
        slab[pl.ds(pl.multiple_of(i * H, 16), D)] = acc
        return carry

    lax.fori_loop(0, _BPW, body, 0)

    cp.wait()
    cs.wait()
    for i in range(_BPW):
        slab[pl.ds(i * H + D, D)] = prow[i]
        slab[pl.ds(i * H + 2 * D, D)] = srow[i]

    pltpu.sync_copy(slab, score_hbm.at[pl.ds(base * H, _BPW * H)])


@functools.cache
def _sc_embed():
    mesh = plsc.VectorSubcoreMesh(core_axis_name="c", subcore_axis_name="s",
                                  num_cores=_NC, num_subcores=_NS)
    return pl.kernel(
        _sc_embed_body,
        mesh=mesh,
        out_type=jax.ShapeDtypeStruct((B * H,), jnp.float32),
        scratch_types=[
            pltpu.VMEM((_BPW * L,), jnp.int32),
            pltpu.VMEM((L, D), jnp.float32),
            pltpu.VMEM((_BPW * H,), jnp.float32),
            pltpu.VMEM((_BPW,), jnp.int32),
            pltpu.VMEM((_BPW,), jnp.int32),
            pltpu.VMEM((_BPW, D), jnp.float32),
            pltpu.VMEM((_BPW, D), jnp.float32),
            pltpu.SemaphoreType.DMA,
        ],
        compiler_params=pltpu.CompilerParams(use_tc_tiling_on_sc=False),
    )


_BB = 32


def _tc_body(x_ref, w1_ref, b1_ref, w2_ref, b2_ref, w3_ref, b3_ref, out_ref):
    h1 = jnp.maximum(
        jnp.dot(x_ref[...], w1_ref[...],
                preferred_element_type=jnp.float32) + b1_ref[...], 0.0)
    h2 = jnp.maximum(
        jnp.dot(h1, w2_ref[...],
                preferred_element_type=jnp.float32) + b2_ref[...], 0.0)
    out_ref[...] = jnp.dot(h2, w3_ref[...],
                           preferred_element_type=jnp.float32) + b3_ref[...]


def _tc_head(score, W1, b1, W2, b2, W3, b3):
    V = W3.shape[1]
    return pl.pallas_call(
        _tc_body,
        grid=(B // _BB,),
        in_specs=[
            pl.BlockSpec((_BB, H), lambda i: (i, 0)),
            pl.BlockSpec((H, H), lambda i: (0, 0)),
            pl.BlockSpec((1, H), lambda i: (0, 0)),
            pl.BlockSpec((H, H), lambda i: (0, 0)),
            pl.BlockSpec((1, H), lambda i: (0, 0)),
            pl.BlockSpec((H, V), lambda i: (0, 0)),
            pl.BlockSpec((1, V), lambda i: (0, 0)),
        ],
        out_specs=pl.BlockSpec((_BB, V), lambda i: (i, 0)),
        out_shape=jax.ShapeDtypeStruct((B, V), jnp.float32),
        compiler_params=pltpu.CompilerParams(
            dimension_semantics=("arbitrary",)),
    )(score, W1, b1, W2, b2, W3, b3)


def kernel(content, pinyin, stroke, word_table, py_table, stroke_table,
           W1, b1, W2, b2, W3, b3):
    score = _sc_embed()(content.reshape(-1), pinyin, stroke,
                        word_table, py_table, stroke_table).reshape(B, H)
    return _tc_head(score, W1, b1.reshape(1, H), W2, b2.reshape(1, H),
                    W3, b3.reshape(1, -1))


# speedup vs baseline: 1.0000x; 1.0000x over previous
import functools

import jax
import jax.numpy as jnp
from jax import lax
from jax.experimental import pallas as pl
from jax.experimental.pallas import tpu as pltpu
from jax.experimental.pallas import tpu_sc as plsc

B = 1024
L = 200
D = 16
H = 3 * D

_NC = 2
_NS = 16
_NW = _NC * _NS
_BPW = B // _NW


def _sc_embed_body(content_hbm, pinyin_hbm, stroke_hbm, wt_hbm, pt_hbm, st_hbm,
                   score_hbm, cidx, rows, slab, pidx, sidx, prow, srow, sem):
    wid = lax.axis_index("s") * _NC + lax.axis_index("c")
    base = wid * _BPW
    pltpu.sync_copy(content_hbm.at[pl.ds(base * L, _BPW * L)], cidx)
    pltpu.sync_copy(pinyin_hbm.at[pl.ds(base, _BPW)], pidx)
    pltpu.sync_copy(stroke_hbm.at[pl.ds(base, _BPW)], sidx)
    cp = pltpu.async_copy(pt_hbm.at[pidx], prow, sem)
    cs = pltpu.async_copy(st_hbm.at[sidx], srow, sem)

    def body(i, carry):
        off = pl.multiple_of(i * L, 8)
        c1 = pltpu.async_copy(
            wt_hbm.at[cidx.at[pl.ds(off, 128)]], rows.at[pl.ds(0, 128)], sem)
        c2 = pltpu.async_copy(
            wt_hbm.at[cidx.at[pl.ds(off + 128, L - 128)]],
            rows.at[pl.ds(128, L - 128)], sem)
        c1.wait()
        c2.wait()
        a0, a1, a2, a3 = rows[0], rows[1], rows[2], rows[3]
        for j in range(4, L, 4):
            a0 += rows[j]
            a1 += rows[j + 1]
            a2 += rows[j + 2]
            a3 += rows[j + 3]
        acc = (a0 + a1) + (a2 + a3)
        slab[pl.ds(pl.multiple_of(i * H, 16), D)] = acc
        return carry

    lax.fori_loop(0, _BPW, body, 0)

    cp.wait()
    cs.wait()
    for i in range(_BPW):
        slab[pl.ds(i * H + D, D)] = prow[i]
        slab[pl.ds(i * H + 2 * D, D)] = srow[i]

    pltpu.sync_copy(slab, score_hbm.at[pl.ds(base * H, _BPW * H)])


@functools.cache
def _sc_embed():
    mesh = plsc.VectorSubcoreMesh(core_axis_name="c", subcore_axis_name="s",
                                  num_cores=_NC, num_subcores=_NS)
    return pl.kernel(
        _sc_embed_body,
        mesh=mesh,
        out_type=jax.ShapeDtypeStruct((B * H,), jnp.float32),
        scratch_types=[
            pltpu.VMEM((_BPW * L,), jnp.int32),
            pltpu.VMEM((L, D), jnp.float32),
            pltpu.VMEM((_BPW * H,), jnp.float32),
            pltpu.VMEM((_BPW,), jnp.int32),
            pltpu.VMEM((_BPW,), jnp.int32),
            pltpu.VMEM((_BPW, D), jnp.float32),
            pltpu.VMEM((_BPW, D), jnp.float32),
            pltpu.SemaphoreType.DMA,
        ],
        compiler_params=pltpu.CompilerParams(use_tc_tiling_on_sc=False),
    )


_BB = 32


def _tc_body(x_ref, w1_ref, b1_ref, w2_ref, b2_ref, w3_ref, b3_ref, out_ref):
    h1 = jnp.maximum(
        jnp.dot(x_ref[...], w1_ref[...],
                preferred_element_type=jnp.float32) + b1_ref[...], 0.0)
    h2 = jnp.maximum(
        jnp.dot(h1, w2_ref[...],
                preferred_element_type=jnp.float32) + b2_ref[...], 0.0)
    out_ref[...] = jnp.dot(h2, w3_ref[...],
                           preferred_element_type=jnp.float32) + b3_ref[...]


def _tc_head(score, W1, b1, W2, b2, W3, b3):
    V = W3.shape[1]
    return pl.pallas_call(
        _tc_body,
        grid=(B // _BB,),
        in_specs=[
            pl.BlockSpec((_BB, H), lambda i: (i, 0)),
            pl.BlockSpec((H, H), lambda i: (0, 0)),
            pl.BlockSpec((1, H), lambda i: (0, 0)),
            pl.BlockSpec((H, H), lambda i: (0, 0)),
            pl.BlockSpec((1, H), lambda i: (0, 0)),
            pl.BlockSpec((H, V), lambda i: (0, 0)),
            pl.BlockSpec((1, V), lambda i: (0, 0)),
        ],
        out_specs=pl.BlockSpec((_BB, V), lambda i: (i, 0)),
        out_shape=jax.ShapeDtypeStruct((B, V), jnp.float32),
        compiler_params=pltpu.CompilerParams(
            dimension_semantics=("parallel",)),
    )(score, W1, b1, W2, b2, W3, b3)


def kernel(content, pinyin, stroke, word_table, py_table, stroke_table,
           W1, b1, W2, b2, W3, b3):
    score = _sc_embed()(content.reshape(-1), pinyin, stroke,
                        word_table, py_table, stroke_table).reshape(B, H)
    return _tc_head(score, W1, b1.reshape(1, H), W2, b2.reshape(1, H),
                    W3, b3.reshape(1, -1))


# speedup vs baseline: 1.1380x; 1.1380x over previous
import functools

import jax
import jax.numpy as jnp
from jax import lax
from jax.experimental import pallas as pl
from jax.experimental.pallas import tpu as pltpu
from jax.experimental.pallas import tpu_sc as plsc

B = 1024
L = 200
D = 16
H = 3 * D

_NC = 2
_NS = 16
_NW = _NC * _NS
_BPW = B // _NW


def _sc_embed_body(content_hbm, pinyin_hbm, stroke_hbm, wt_hbm, pt_hbm, st_hbm,
                   score_hbm, cidx, rows, slab, pidx, sidx, prow, srow, sem):
    wid = lax.axis_index("s") * _NC + lax.axis_index("c")
    base = wid * _BPW
    pltpu.sync_copy(content_hbm.at[pl.ds(base * L, _BPW * L)], cidx)
    pltpu.sync_copy(pinyin_hbm.at[pl.ds(base, _BPW)], pidx)
    pltpu.sync_copy(stroke_hbm.at[pl.ds(base, _BPW)], sidx)
    cp = pltpu.async_copy(pt_hbm.at[pidx], prow, sem)
    cs = pltpu.async_copy(st_hbm.at[sidx], srow, sem)

    def body(i, carry):
        off = pl.multiple_of(i * L, 8)
        c1 = pltpu.async_copy(
            wt_hbm.at[cidx.at[pl.ds(off, 128)]], rows.at[pl.ds(0, 128)], sem)
        c2 = pltpu.async_copy(
            wt_hbm.at[cidx.at[pl.ds(off + 128, L - 128)]],
            rows.at[pl.ds(128, L - 128)], sem)
        c1.wait()
        c2.wait()
        a0, a1, a2, a3 = rows[0], rows[1], rows[2], rows[3]
        for j in range(4, L, 4):
            a0 += rows[j]
            a1 += rows[j + 1]
            a2 += rows[j + 2]
            a3 += rows[j + 3]
        acc = (a0 + a1) + (a2 + a3)
        slab[pl.ds(pl.multiple_of(i * H, 16), D)] = acc
        return carry

    lax.fori_loop(0, _BPW, body, 0)

    cp.wait()
    cs.wait()
    for i in range(_BPW):
        slab[pl.ds(i * H + D, D)] = prow[i]
        slab[pl.ds(i * H + 2 * D, D)] = srow[i]

    pltpu.sync_copy(slab, score_hbm.at[pl.ds(base * H, _BPW * H)])


@functools.cache
def _sc_embed():
    mesh = plsc.VectorSubcoreMesh(core_axis_name="c", subcore_axis_name="s",
                                  num_cores=_NC, num_subcores=_NS)
    return pl.kernel(
        _sc_embed_body,
        mesh=mesh,
        out_type=jax.ShapeDtypeStruct((B * H,), jnp.float32),
        scratch_types=[
            pltpu.VMEM((_BPW * L,), jnp.int32),
            pltpu.VMEM((L, D), jnp.float32),
            pltpu.VMEM((_BPW * H,), jnp.float32),
            pltpu.VMEM((_BPW,), jnp.int32),
            pltpu.VMEM((_BPW,), jnp.int32),
            pltpu.VMEM((_BPW, D), jnp.float32),
            pltpu.VMEM((_BPW, D), jnp.float32),
            pltpu.SemaphoreType.DMA,
        ],
        compiler_params=pltpu.CompilerParams(use_tc_tiling_on_sc=False),
    )


_BB = 32


def _tc_body(x_ref, w1_ref, b1_ref, w2_ref, b2_ref, w3_ref, b3_ref, out_ref):
    h1 = jnp.maximum(
        jnp.dot(x_ref[...], w1_ref[...],
                preferred_element_type=jnp.float32) + b1_ref[...], 0.0)
    h2 = jnp.maximum(
        jnp.dot(h1, w2_ref[...],
                preferred_element_type=jnp.float32) + b2_ref[...], 0.0)
    out_ref[:, :w3_ref.shape[1]] = jnp.dot(
        h2, w3_ref[...], preferred_element_type=jnp.float32) + b3_ref[...]


def _tc_head(score, W1, b1, W2, b2, W3, b3):
    V = W3.shape[1]
    Vp = (V + 127) // 128 * 128
    out = pl.pallas_call(
        _tc_body,
        grid=(B // _BB,),
        in_specs=[
            pl.BlockSpec((_BB, H), lambda i: (i, 0)),
            pl.BlockSpec((H, H), lambda i: (0, 0)),
            pl.BlockSpec((1, H), lambda i: (0, 0)),
            pl.BlockSpec((H, H), lambda i: (0, 0)),
            pl.BlockSpec((1, H), lambda i: (0, 0)),
            pl.BlockSpec((H, V), lambda i: (0, 0)),
            pl.BlockSpec((1, V), lambda i: (0, 0)),
        ],
        out_specs=pl.BlockSpec((_BB, Vp), lambda i: (i, 0)),
        out_shape=jax.ShapeDtypeStruct((B, Vp), jnp.float32),
        compiler_params=pltpu.CompilerParams(
            dimension_semantics=("parallel",)),
    )(score, W1, b1, W2, b2, W3, b3)
    return out[:, :V]


def kernel(content, pinyin, stroke, word_table, py_table, stroke_table,
           W1, b1, W2, b2, W3, b3):
    score = _sc_embed()(content.reshape(-1), pinyin, stroke,
                        word_table, py_table, stroke_table).reshape(B, H)
    return _tc_head(score, W1, b1.reshape(1, H), W2, b2.reshape(1, H),
                    W3, b3.reshape(1, -1))
